# baseline (speedup 1.0000x reference)
import functools

import jax
import jax.numpy as jnp
from jax import lax
from jax.experimental import pallas as pl
from jax.experimental.pallas import tpu as pltpu
from jax.experimental.pallas import tpu_sc as plsc

N, C, H, W = 2, 96, 224, 224
HW = H * W
LANES = 16
NSUB = 16
CHUNK = HW // NSUB
GROUPS = CHUNK // LANES
HB = 16


def _stage_a_body(f_ref, w_ref, tgt_ref, b_ref):
    h = pl.program_id(1)
    f = f_ref[0].reshape(C, HB * W)
    w = w_ref[...]
    r = jax.lax.dot_general(w, f, (((1,), (0,)), ((), ())),
                            preferred_element_type=jnp.float32)
    r = r.reshape(8, HB, W)
    c0, c1 = r[0], r[1]
    s0 = jnp.maximum(r[2], 0.0)
    s1 = jnp.maximum(r[3], 0.0)
    p = jax.nn.sigmoid(r[4])
    off0 = c0 * s0
    off1 = c1 * s1
    i = (h * HB).astype(jnp.float32) + lax.broadcasted_iota(
        jnp.int32, (HB, W), 0).astype(jnp.float32)
    j = lax.broadcasted_iota(jnp.int32, (HB, W), 1).astype(jnp.float32)
    t0 = jnp.minimum(jnp.round(i + off0), float(H - 1))
    t1 = jnp.minimum(jnp.round(j + off1), float(W - 1))
    tf = t0 * W + t1
    tf = jnp.where(tf < 0, tf + HW, tf)
    tgt_ref[0] = tf.astype(jnp.int32)
    b_ref[0] = p * (1.0 - p)


def _stage_a(feature, w5):
    grid = (N, H // HB)
    return pl.pallas_call(
        _stage_a_body,
        grid=grid,
        in_specs=[
            pl.BlockSpec((1, C, HB, W), lambda n, h: (n, 0, h, 0)),
            pl.BlockSpec((8, C), lambda n, h: (0, 0)),
        ],
        out_specs=[
            pl.BlockSpec((1, HB, W), lambda n, h: (n, h, 0)),
            pl.BlockSpec((1, HB, W), lambda n, h: (n, h, 0)),
        ],
        out_shape=[
            jax.ShapeDtypeStruct((N, H, W), jnp.int32),
            jax.ShapeDtypeStruct((N, H, W), jnp.float32),
        ],
    )(feature, w5)


def _transpose_addr(k):
    q = ((k >> 5) * 9363) >> 16
    r = k - q * W
    return r * W + q


def _stage_b_body(tgt_hbm, b_hbm, combo_hbm, tgt_v, b_v, ptab, sh, acc, tmp2,
                  outv, msem0, msem1):
    n = lax.axis_index("c")
    s = lax.axis_index("s")
    base = s * CHUNK
    lanes = lax.iota(jnp.int32, LANES)
    zeros_i = jnp.zeros((LANES,), jnp.int32)
    msem = (msem0, msem1)

    @plsc.parallel_loop(0, GROUPS * NSUB, unroll=8)
    def _zero(q):
        ptab[pl.ds(q * LANES, LANES)] = zeros_i

    pltpu.sync_copy(tgt_hbm.at[pl.ds(n * HW + base, CHUNK)], tgt_v)

    @pl.loop(0, GROUPS)
    def _scan(q):
        t = tgt_v[pl.ds(q * LANES, LANES)]
        jg1 = base + q * LANES + lanes + 1
        valid = t >= 0
        tsafe = jnp.maximum(t, 0)

        def _body(_, need):
            plsc.store_scatter(ptab, [tsafe], jg1, mask=need)
            w = plsc.load_gather(ptab, [tsafe])
            return valid & (w < jg1)

        lax.fori_loop(0, LANES, _body, valid, unroll=False)

    pltpu.sync_copy(ptab, sh.at[pl.ds(s * HW, HW)])
    plsc.subcore_barrier()

    pltpu.sync_copy(sh.at[pl.ds(base, CHUNK)], acc)
    mdesc = pltpu.async_copy(sh.at[pl.ds(HW + base, CHUNK)],
                             tmp2.at[pl.ds(0, CHUNK)], msem0)
    for i in range(1, NSUB):
        par = (i - 1) & 1
        mdesc.wait()
        if i + 1 < NSUB:
            mdesc = pltpu.async_copy(
                sh.at[pl.ds((i + 1) * HW + base, CHUNK)],
                tmp2.at[pl.ds((1 - par) * CHUNK, CHUNK)], msem[1 - par])

        @plsc.parallel_loop(0, GROUPS, unroll=8)
        def _merge(q):
            sl = pl.ds(q * LANES, LANES)
            acc[sl] = jnp.maximum(acc[sl], tmp2[pl.ds(par * CHUNK + q * LANES,
                                                      LANES)])

    pltpu.sync_copy(b_hbm.at[pl.ds(n * HW + base, CHUNK)], b_v)

    @plsc.parallel_loop(0, GROUPS, unroll=4)
    def _final(q):
        sl = pl.ds(q * LANES, LANES)
        av = acc[sl]
        k = base + q * LANES + lanes
        jw = jnp.where(av == 0, k, av - 1)
        g = _transpose_addr(jw).astype(jnp.uint32)
        bb = (plsc.bitcast(b_v[sl], jnp.uint32) + 0x8000) >> 16
        outv[sl] = plsc.bitcast((g << 16) | bb, jnp.int32)

    pltpu.sync_copy(outv, combo_hbm.at[pl.ds(n * HW + base, CHUNK)])


def _stage_b(tgt, b):
    mesh = plsc.VectorSubcoreMesh(core_axis_name="c", subcore_axis_name="s")
    return pl.kernel(
        _stage_b_body,
        out_type=jax.ShapeDtypeStruct((N * HW,), jnp.int32),
        mesh=mesh,
        compiler_params=pltpu.CompilerParams(needs_layout_passes=False),
        scratch_types=[
            pltpu.VMEM((CHUNK,), jnp.int32),
            pltpu.VMEM((CHUNK,), jnp.float32),
            pltpu.VMEM((HW,), jnp.int32),
            pltpu.VMEM_SHARED((NSUB * HW,), jnp.int32),
            pltpu.VMEM((CHUNK,), jnp.int32),
            pltpu.VMEM((2 * CHUNK,), jnp.int32),
            pltpu.VMEM((CHUNK,), jnp.int32),
            pltpu.SemaphoreType.DMA,
            pltpu.SemaphoreType.DMA,
        ],
    )(tgt, b)


def _stage_c_body(feat_hbm, combo_hbm, out_hbm, pv0, pv1, cv2, o02, o12,
                  cvsem, osem0, osem1):
    n = lax.axis_index("c")
    s = lax.axis_index("s")
    osem = (osem0, osem1)

    for t in range(3):
        p0 = n * C + s * 6 + 2 * t
        p1 = p0 + 1
        pltpu.sync_copy(feat_hbm.at[pl.ds(p0 * HW, HW)], pv0)
        pltpu.sync_copy(feat_hbm.at[pl.ds(p1 * HW, HW)], pv1)

        out_descs = [None] * NSUB
        cv_desc = pltpu.async_copy(
            combo_hbm.at[pl.ds(n * HW, CHUNK)], cv2.at[pl.ds(0, CHUNK)], cvsem)
        for ch in range(NSUB):
            par = ch & 1
            off = ch * CHUNK
            cv_desc.wait()
            if ch + 1 < NSUB:
                cv_desc = pltpu.async_copy(
                    combo_hbm.at[pl.ds(n * HW + (ch + 1) * CHUNK, CHUNK)],
                    cv2.at[pl.ds((1 - par) * CHUNK, CHUNK)], cvsem)
            if ch >= 2:
                out_descs[ch - 2][0].wait()
                out_descs[ch - 2][1].wait()

            @plsc.parallel_loop(0, GROUPS, unroll=8)
            def _px(q):
                sl = pl.ds(par * CHUNK + q * LANES, LANES)
                cu = plsc.bitcast(cv2[sl], jnp.uint32)
                idx = (cu >> 16).astype(jnp.int32)
                bv = plsc.bitcast(cu << 16, jnp.float32)
                f0 = pv0[pl.ds(off + q * LANES, LANES)]
                g0 = plsc.load_gather(pv0, [idx])
                o02[sl] = f0 + bv * (g0 - f0)
                f1 = pv1[pl.ds(off + q * LANES, LANES)]
                g1 = plsc.load_gather(pv1, [idx])
                o12[sl] = f1 + bv * (g1 - f1)

            d0 = pltpu.async_copy(o02.at[pl.ds(par * CHUNK, CHUNK)],
                                  out_hbm.at[pl.ds(p0 * HW + off, CHUNK)],
                                  osem[par])
            d1 = pltpu.async_copy(o12.at[pl.ds(par * CHUNK, CHUNK)],
                                  out_hbm.at[pl.ds(p1 * HW + off, CHUNK)],
                                  osem[par])
            out_descs[ch] = (d0, d1)
        for ch in (NSUB - 2, NSUB - 1):
            out_descs[ch][0].wait()
            out_descs[ch][1].wait()


def _stage_c(feat_flat, combo):
    mesh = plsc.VectorSubcoreMesh(core_axis_name="c", subcore_axis_name="s")
    return pl.kernel(
        _stage_c_body,
        out_type=jax.ShapeDtypeStruct((N * C * HW,), jnp.float32),
        mesh=mesh,
        compiler_params=pltpu.CompilerParams(needs_layout_passes=False),
        scratch_types=[
            pltpu.VMEM((HW,), jnp.float32),
            pltpu.VMEM((HW,), jnp.float32),
            pltpu.VMEM((2 * CHUNK,), jnp.int32),
            pltpu.VMEM((2 * CHUNK,), jnp.float32),
            pltpu.VMEM((2 * CHUNK,), jnp.float32),
            pltpu.SemaphoreType.DMA,
            pltpu.SemaphoreType.DMA,
            pltpu.SemaphoreType.DMA,
        ],
    )(feat_flat, combo)


@jax.jit
def kernel(feature, W_center, W_step, W_prob):
    w5 = jnp.concatenate(
        [W_center, W_step, W_prob, jnp.zeros((3, C), jnp.float32)], axis=0)
    tgt, b = _stage_a(feature, w5)
    combo = _stage_b(tgt.reshape(N * HW), b.reshape(N * HW))
    out = _stage_c(feature.reshape(N * C * HW), combo)
    return out.reshape(N, C, H, W)


# speedup vs baseline: 1.0166x; 1.0166x over previous
import functools

import jax
import jax.numpy as jnp
from jax import lax
from jax.experimental import pallas as pl
from jax.experimental.pallas import tpu as pltpu
from jax.experimental.pallas import tpu_sc as plsc

N, C, H, W = 2, 96, 224, 224
HW = H * W
LANES = 16
NSUB = 16
CHUNK = HW // NSUB
GROUPS = CHUNK // LANES
HB = 16


def _stage_a_body(f_ref, w_ref, tgt_ref, b_ref):
    h = pl.program_id(1)
    f = f_ref[0].reshape(C, HB * W)
    w = w_ref[...]
    r = jax.lax.dot_general(w, f, (((1,), (0,)), ((), ())),
                            preferred_element_type=jnp.float32)
    r = r.reshape(8, HB, W)
    c0, c1 = r[0], r[1]
    s0 = jnp.maximum(r[2], 0.0)
    s1 = jnp.maximum(r[3], 0.0)
    p = jax.nn.sigmoid(r[4])
    off0 = c0 * s0
    off1 = c1 * s1
    i = (h * HB).astype(jnp.float32) + lax.broadcasted_iota(
        jnp.int32, (HB, W), 0).astype(jnp.float32)
    j = lax.broadcasted_iota(jnp.int32, (HB, W), 1).astype(jnp.float32)
    t0 = jnp.minimum(jnp.round(i + off0), float(H - 1))
    t1 = jnp.minimum(jnp.round(j + off1), float(W - 1))
    tf = t0 * W + t1
    tf = jnp.where(tf < 0, tf + HW, tf)
    tgt_ref[0] = tf.astype(jnp.int32)
    b_ref[0] = p * (1.0 - p)


def _stage_a(feature, w5):
    grid = (N, H // HB)
    return pl.pallas_call(
        _stage_a_body,
        grid=grid,
        in_specs=[
            pl.BlockSpec((1, C, HB, W), lambda n, h: (n, 0, h, 0)),
            pl.BlockSpec((8, C), lambda n, h: (0, 0)),
        ],
        out_specs=[
            pl.BlockSpec((1, HB, W), lambda n, h: (n, h, 0)),
            pl.BlockSpec((1, HB, W), lambda n, h: (n, h, 0)),
        ],
        out_shape=[
            jax.ShapeDtypeStruct((N, H, W), jnp.int32),
            jax.ShapeDtypeStruct((N, H, W), jnp.float32),
        ],
    )(feature, w5)


def _transpose_addr(k):
    q = ((k >> 5) * 9363) >> 16
    r = k - q * W
    return r * W + q


def _stage_b_body(tgt_hbm, b_hbm, combo_hbm, tgt_v, b_v, ptab, sh, acc, tmp2,
                  outv, msem0, msem1):
    n = lax.axis_index("c")
    s = lax.axis_index("s")
    base = s * CHUNK
    lanes = lax.iota(jnp.int32, LANES)
    zeros_i = jnp.zeros((LANES,), jnp.int32)
    msem = (msem0, msem1)

    @plsc.parallel_loop(0, GROUPS * NSUB, unroll=8)
    def _zero(q):
        ptab[pl.ds(q * LANES, LANES)] = zeros_i

    pltpu.sync_copy(tgt_hbm.at[pl.ds(n * HW + base, CHUNK)], tgt_v)

    @pl.loop(0, GROUPS)
    def _scan(q):
        t = tgt_v[pl.ds(q * LANES, LANES)]
        jg1 = base + q * LANES + lanes + 1
        valid = t >= 0
        tsafe = jnp.maximum(t, 0)

        def _body(_, need):
            plsc.store_scatter(ptab, [tsafe], jg1, mask=need)
            w = plsc.load_gather(ptab, [tsafe])
            return valid & (w < jg1)

        lax.fori_loop(0, LANES, _body, valid, unroll=False)

    pltpu.sync_copy(ptab, sh.at[pl.ds(s * HW, HW)])
    plsc.subcore_barrier()

    pltpu.sync_copy(sh.at[pl.ds(base, CHUNK)], acc)
    mdesc = pltpu.async_copy(sh.at[pl.ds(HW + base, CHUNK)],
                             tmp2.at[pl.ds(0, CHUNK)], msem0)
    for i in range(1, NSUB):
        par = (i - 1) & 1
        mdesc.wait()
        if i + 1 < NSUB:
            mdesc = pltpu.async_copy(
                sh.at[pl.ds((i + 1) * HW + base, CHUNK)],
                tmp2.at[pl.ds((1 - par) * CHUNK, CHUNK)], msem[1 - par])

        @plsc.parallel_loop(0, GROUPS, unroll=8)
        def _merge(q):
            sl = pl.ds(q * LANES, LANES)
            acc[sl] = jnp.maximum(acc[sl], tmp2[pl.ds(par * CHUNK + q * LANES,
                                                      LANES)])

    pltpu.sync_copy(b_hbm.at[pl.ds(n * HW + base, CHUNK)], b_v)

    @plsc.parallel_loop(0, GROUPS, unroll=4)
    def _final(q):
        sl = pl.ds(q * LANES, LANES)
        av = acc[sl]
        k = base + q * LANES + lanes
        jw = jnp.where(av == 0, k, av - 1)
        g = _transpose_addr(jw).astype(jnp.uint32)
        bb = (plsc.bitcast(b_v[sl], jnp.uint32) + 0x8000) >> 16
        outv[sl] = plsc.bitcast((g << 16) | bb, jnp.int32)

    pltpu.sync_copy(outv, combo_hbm.at[pl.ds(n * HW + base, CHUNK)])


def _stage_b(tgt, b):
    mesh = plsc.VectorSubcoreMesh(core_axis_name="c", subcore_axis_name="s")
    return pl.kernel(
        _stage_b_body,
        out_type=jax.ShapeDtypeStruct((N * HW,), jnp.int32),
        mesh=mesh,
        compiler_params=pltpu.CompilerParams(needs_layout_passes=False),
        scratch_types=[
            pltpu.VMEM((CHUNK,), jnp.int32),
            pltpu.VMEM((CHUNK,), jnp.float32),
            pltpu.VMEM((HW,), jnp.int32),
            pltpu.VMEM_SHARED((NSUB * HW,), jnp.int32),
            pltpu.VMEM((CHUNK,), jnp.int32),
            pltpu.VMEM((2 * CHUNK,), jnp.int32),
            pltpu.VMEM((CHUNK,), jnp.int32),
            pltpu.SemaphoreType.DMA,
            pltpu.SemaphoreType.DMA,
        ],
    )(tgt, b)


OUTQ = HW // 4
QGROUPS = OUTQ // LANES


def _stage_c_body(feat_hbm, combo_hbm, out_hbm, pv, cv, oq2, osem0, osem1):
    n = lax.axis_index("c")
    s = lax.axis_index("s")
    osem = (osem0, osem1)

    pltpu.sync_copy(combo_hbm.at[pl.ds(n * HW, HW)], cv)

    for t in range(6):
        plane = n * C + s * 6 + t
        pltpu.sync_copy(feat_hbm.at[pl.ds(plane * HW, HW)], pv)
        out_descs = [None] * 4
        for ch in range(4):
            par = ch & 1
            off = ch * OUTQ
            if ch >= 2 or t > 0:
                prev = out_descs[ch - 2] if ch >= 2 else prev_descs[ch + 2]
                prev.wait()

            @plsc.parallel_loop(0, QGROUPS, unroll=4)
            def _px(q):
                cu = plsc.bitcast(cv[pl.ds(off + q * LANES, LANES)],
                                  jnp.uint32)
                idx = (cu >> 16).astype(jnp.int32)
                bv = plsc.bitcast(cu << 16, jnp.float32)
                f = pv[pl.ds(off + q * LANES, LANES)]
                g = plsc.load_gather(pv, [idx])
                oq2[pl.ds(par * OUTQ + q * LANES, LANES)] = f + bv * (g - f)

            out_descs[ch] = pltpu.async_copy(
                oq2.at[pl.ds(par * OUTQ, OUTQ)],
                out_hbm.at[pl.ds(plane * HW + off, OUTQ)], osem[par])
        prev_descs = out_descs
    out_descs[2].wait()
    out_descs[3].wait()


def _stage_c(feat_flat, combo):
    mesh = plsc.VectorSubcoreMesh(core_axis_name="c", subcore_axis_name="s")
    return pl.kernel(
        _stage_c_body,
        out_type=jax.ShapeDtypeStruct((N * C * HW,), jnp.float32),
        mesh=mesh,
        compiler_params=pltpu.CompilerParams(needs_layout_passes=False),
        scratch_types=[
            pltpu.VMEM((HW,), jnp.float32),
            pltpu.VMEM((HW,), jnp.int32),
            pltpu.VMEM((2 * OUTQ,), jnp.float32),
            pltpu.SemaphoreType.DMA,
            pltpu.SemaphoreType.DMA,
        ],
    )(feat_flat, combo)


@jax.jit
def kernel(feature, W_center, W_step, W_prob):
    w5 = jnp.concatenate(
        [W_center, W_step, W_prob, jnp.zeros((3, C), jnp.float32)], axis=0)
    tgt, b = _stage_a(feature, w5)
    combo = _stage_b(tgt.reshape(N * HW), b.reshape(N * HW))
    out = _stage_c(feature.reshape(N * C * HW), combo)
    return out.reshape(N, C, H, W)


# speedup vs baseline: 1.6214x; 1.5949x over previous
import functools

import jax
import jax.numpy as jnp
from jax import lax
from jax.experimental import pallas as pl
from jax.experimental.pallas import tpu as pltpu
from jax.experimental.pallas import tpu_sc as plsc

N, C, H, W = 2, 96, 224, 224
HW = H * W
LANES = 16
NSUB = 16
CHUNK = HW // NSUB
GROUPS = CHUNK // LANES
HB = 16
WPAD = W + 1


def _stage_a_body(f_ref, w_ref, tgt_ref, b_ref, fp_ref):
    h = pl.program_id(1)
    f = f_ref[0].reshape(C, HB * W)
    w = w_ref[...]
    r = jax.lax.dot_general(w, f, (((1,), (0,)), ((), ())),
                            preferred_element_type=jnp.float32)
    r = r.reshape(8, HB, W)
    c0, c1 = r[0], r[1]
    s0 = jnp.maximum(r[2], 0.0)
    s1 = jnp.maximum(r[3], 0.0)
    p = jax.nn.sigmoid(r[4])
    off0 = c0 * s0
    off1 = c1 * s1
    i = (h * HB).astype(jnp.float32) + lax.broadcasted_iota(
        jnp.int32, (HB, W), 0).astype(jnp.float32)
    j = lax.broadcasted_iota(jnp.int32, (HB, W), 1).astype(jnp.float32)
    t0 = jnp.minimum(jnp.round(i + off0), float(H - 1))
    t1 = jnp.minimum(jnp.round(j + off1), float(W - 1))
    tf = t0 * W + t1
    tf = jnp.where(tf < 0, tf + HW, tf)
    tgt_ref[0] = tf.astype(jnp.int32)
    b_ref[0] = p * (1.0 - p)
    fp_ref[0] = jnp.concatenate(
        [f_ref[0], jnp.zeros((C, HB, 1), jnp.float32)], axis=2)


def _stage_a(feature, w5):
    grid = (N, H // HB)
    return pl.pallas_call(
        _stage_a_body,
        grid=grid,
        in_specs=[
            pl.BlockSpec((1, C, HB, W), lambda n, h: (n, 0, h, 0)),
            pl.BlockSpec((8, C), lambda n, h: (0, 0)),
        ],
        out_specs=[
            pl.BlockSpec((1, HB, W), lambda n, h: (n, h, 0)),
            pl.BlockSpec((1, HB, W), lambda n, h: (n, h, 0)),
            pl.BlockSpec((1, C, HB, WPAD), lambda n, h: (n, 0, h, 0)),
        ],
        out_shape=[
            jax.ShapeDtypeStruct((N, H, W), jnp.int32),
            jax.ShapeDtypeStruct((N, H, W), jnp.float32),
            jax.ShapeDtypeStruct((N, C, H, WPAD), jnp.float32),
        ],
    )(feature, w5)


def _transpose_addr(k):
    q = ((k >> 5) * 9363) >> 16
    r = k - q * W
    return r * WPAD + q


def _stage_b_body(tgt_hbm, b_hbm, combo_hbm, tgt_v, b_v, ptab, sh, acc, tmp2,
                  outv, msem0, msem1):
    n = lax.axis_index("c")
    s = lax.axis_index("s")
    base = s * CHUNK
    lanes = lax.iota(jnp.int32, LANES)
    zeros_i = jnp.zeros((LANES,), jnp.int32)
    msem = (msem0, msem1)

    @plsc.parallel_loop(0, GROUPS * NSUB, unroll=8)
    def _zero(q):
        ptab[pl.ds(q * LANES, LANES)] = zeros_i

    pltpu.sync_copy(tgt_hbm.at[pl.ds(n * HW + base, CHUNK)], tgt_v)

    @pl.loop(0, GROUPS)
    def _scan(q):
        t = tgt_v[pl.ds(q * LANES, LANES)]
        jg1 = base + q * LANES + lanes + 1
        valid = t >= 0
        tsafe = jnp.maximum(t, 0)

        def _body(_, need):
            plsc.store_scatter(ptab, [tsafe], jg1, mask=need)
            w = plsc.load_gather(ptab, [tsafe])
            return valid & (w < jg1)

        lax.fori_loop(0, LANES, _body, valid, unroll=False)

    pltpu.sync_copy(ptab, sh.at[pl.ds(s * HW, HW)])
    plsc.subcore_barrier()

    pltpu.sync_copy(sh.at[pl.ds(base, CHUNK)], acc)
    mdesc = pltpu.async_copy(sh.at[pl.ds(HW + base, CHUNK)],
                             tmp2.at[pl.ds(0, CHUNK)], msem0)
    for i in range(1, NSUB):
        par = (i - 1) & 1
        mdesc.wait()
        if i + 1 < NSUB:
            mdesc = pltpu.async_copy(
                sh.at[pl.ds((i + 1) * HW + base, CHUNK)],
                tmp2.at[pl.ds((1 - par) * CHUNK, CHUNK)], msem[1 - par])

        @plsc.parallel_loop(0, GROUPS, unroll=8)
        def _merge(q):
            sl = pl.ds(q * LANES, LANES)
            acc[sl] = jnp.maximum(acc[sl], tmp2[pl.ds(par * CHUNK + q * LANES,
                                                      LANES)])

    pltpu.sync_copy(b_hbm.at[pl.ds(n * HW + base, CHUNK)], b_v)

    @plsc.parallel_loop(0, GROUPS, unroll=4)
    def _final(q):
        sl = pl.ds(q * LANES, LANES)
        av = acc[sl]
        k = base + q * LANES + lanes
        jw = jnp.where(av == 0, k, av - 1)
        g = _transpose_addr(jw).astype(jnp.uint32)
        bb = (plsc.bitcast(b_v[sl], jnp.uint32) + 0x8000) >> 16
        outv[sl] = plsc.bitcast((g << 16) | bb, jnp.int32)

    pltpu.sync_copy(outv, combo_hbm.at[pl.ds(n * HW + base, CHUNK)])


def _stage_b(tgt, b):
    mesh = plsc.VectorSubcoreMesh(core_axis_name="c", subcore_axis_name="s")
    return pl.kernel(
        _stage_b_body,
        out_type=jax.ShapeDtypeStruct((N * HW,), jnp.int32),
        mesh=mesh,
        compiler_params=pltpu.CompilerParams(needs_layout_passes=False),
        scratch_types=[
            pltpu.VMEM((CHUNK,), jnp.int32),
            pltpu.VMEM((CHUNK,), jnp.float32),
            pltpu.VMEM((HW,), jnp.int32),
            pltpu.VMEM_SHARED((NSUB * HW,), jnp.int32),
            pltpu.VMEM((CHUNK,), jnp.int32),
            pltpu.VMEM((2 * CHUNK,), jnp.int32),
            pltpu.VMEM((CHUNK,), jnp.int32),
            pltpu.SemaphoreType.DMA,
            pltpu.SemaphoreType.DMA,
        ],
    )(tgt, b)


OUTQ = HW // 4
QROWS = H // 4


def _stage_c_body(feat_hbm, combo_hbm, out_hbm, pv, cv, oq2, osem0, osem1):
    n = lax.axis_index("c")
    s = lax.axis_index("s")
    osem = (osem0, osem1)

    pltpu.sync_copy(combo_hbm.at[pl.ds(n * HW, HW)], cv)

    @pl.loop(0, 6)
    def _plane_loop(t):
        plane = n * C + s * 6 + t
        pltpu.sync_copy(feat_hbm.at[pl.ds(plane * H * WPAD, H * WPAD)], pv)
        out_descs = [None] * 4
        for ch in range(4):
            par = ch & 1
            off = ch * OUTQ
            if ch >= 2:
                out_descs[ch - 2].wait()

            @plsc.parallel_loop(0, QROWS, unroll=1)
            def _row(rr):
                r = ch * QROWS + rr
                for gi in range(W // LANES):
                    c0 = gi * LANES
                    cu = plsc.bitcast(cv[pl.ds(r * W + c0, LANES)],
                                      jnp.uint32)
                    idx = (cu >> 16).astype(jnp.int32)
                    bv = plsc.bitcast(cu << 16, jnp.float32)
                    f = pv[pl.ds(r * WPAD + c0, LANES)]
                    g = plsc.load_gather(pv, [idx])
                    oq2[pl.ds(par * OUTQ + rr * W + c0, LANES)] = (
                        f + bv * (g - f))

            out_descs[ch] = pltpu.async_copy(
                oq2.at[pl.ds(par * OUTQ, OUTQ)],
                out_hbm.at[pl.ds(plane * HW + off, OUTQ)], osem[par])
        out_descs[2].wait()
        out_descs[3].wait()


def _stage_c(feat_flat, combo):
    mesh = plsc.VectorSubcoreMesh(core_axis_name="c", subcore_axis_name="s")
    return pl.kernel(
        _stage_c_body,
        out_type=jax.ShapeDtypeStruct((N * C * HW,), jnp.float32),
        mesh=mesh,
        compiler_params=pltpu.CompilerParams(needs_layout_passes=False),
        scratch_types=[
            pltpu.VMEM((H * WPAD,), jnp.float32),
            pltpu.VMEM((HW,), jnp.int32),
            pltpu.VMEM((2 * OUTQ,), jnp.float32),
            pltpu.SemaphoreType.DMA,
            pltpu.SemaphoreType.DMA,
        ],
    )(feat_flat, combo)


@jax.jit
def kernel(feature, W_center, W_step, W_prob):
    w5 = jnp.concatenate(
        [W_center, W_step, W_prob, jnp.zeros((3, C), jnp.float32)], axis=0)
    tgt, b, fpad = _stage_a(feature, w5)
    combo = _stage_b(tgt.reshape(N * HW), b.reshape(N * HW))
    out = _stage_c(fpad.reshape(N * C * H * WPAD), combo)
    return out.reshape(N, C, H, W)
